# Initial kernel scaffold; baseline (speedup 1.0000x reference)
import jax
import jax.numpy as jnp
from jax.experimental import pallas as pl


def kernel(feature, edge_index, W, b):
    raise NotImplementedError("write your pallas kernel here")


# speedup vs baseline: 4.7017x; 4.7017x over previous
import functools

import jax
import jax.numpy as jnp
from jax import lax
from jax.experimental import pallas as pl
from jax.experimental.pallas import tpu as pltpu
from jax.experimental.pallas import tpu_sc as plsc

N_NODES = 10000
D = 128

NC = 2
NS = 16
NW = NC * NS

C = 128
K = 79
EDGES_PAD = NW * K * C

ACC_ROWS = 10240
DUMMY_ROW = N_NODES

@functools.cache
def _build_sc_message_pass():
    mesh = plsc.VectorSubcoreMesh(core_axis_name="c", subcore_axis_name="s")
    return pl.kernel(
        _sc_message_pass_body,
        out_type=jax.ShapeDtypeStruct((NC, N_NODES, D), jnp.float32),
        mesh=mesh,
        scratch_types=[
            pltpu.VMEM((K, C), jnp.int32),
            pltpu.VMEM((K, C), jnp.int32),
            pltpu.VMEM((C, D), jnp.float32),
            pltpu.VMEM_SHARED((ACC_ROWS, D), jnp.float32),
        ],
    )


def _sc_message_pass_body(feat_hbm, src_hbm, dst_hbm, out_hbm,
                          src_v, dst_v, rows_v, acc_sh):
    cid = lax.axis_index("c")
    sid = lax.axis_index("s")
    wid = cid * NS + sid

    @pl.loop(0, C)
    def _(r):
        @pl.loop(0, D, step=16)
        def _(c):
            rows_v.at[pl.ds(r, 1), pl.ds(c, 16)][...] = jnp.zeros(
                (1, 16), jnp.float32)

    @pl.loop(0, ACC_ROWS // (NS * C))
    def _(k):
        pltpu.sync_copy(rows_v, acc_sh.at[pl.ds(sid * (ACC_ROWS // NS) + k * C, C)])

    plsc.subcore_barrier()

    pltpu.sync_copy(src_hbm.at[wid], src_v)
    pltpu.sync_copy(dst_hbm.at[wid], dst_v)

    @pl.loop(0, K)
    def _(j):
        pltpu.sync_copy(feat_hbm.at[src_v.at[j]], rows_v)
        pltpu.sync_copy(rows_v, acc_sh.at[dst_v.at[j]], add=True)

    plsc.subcore_barrier()

    rows_per = 624
    pltpu.sync_copy(acc_sh.at[pl.ds(sid * rows_per, rows_per)],
                    out_hbm.at[cid, pl.ds(sid * rows_per, rows_per)])

    @pl.when(sid == 0)
    def _():
        pltpu.sync_copy(acc_sh.at[pl.ds(NS * rows_per, N_NODES - NS * rows_per)],
                        out_hbm.at[cid, pl.ds(NS * rows_per, N_NODES - NS * rows_per)])


def _tc_linear_body(p_ref, w_ref, b_ref, o_ref):
    h = p_ref[0] + p_ref[1]
    y = lax.dot_general(
        h, w_ref[...],
        dimension_numbers=(((1,), (1,)), ((), ())),
        precision=lax.Precision.HIGHEST,
        preferred_element_type=jnp.float32,
    )
    o_ref[...] = jnp.maximum(y + b_ref[...], 0.0)


def kernel(feature, edge_index, W, b):
    n_edges = edge_index.shape[1]
    pad = EDGES_PAD - n_edges
    src = jnp.concatenate([edge_index[0], jnp.zeros((pad,), jnp.int32)])
    dst = jnp.concatenate(
        [edge_index[1], jnp.full((pad,), DUMMY_ROW, jnp.int32)])
    src3 = src.reshape(NW, K, C)
    dst3 = dst.reshape(NW, K, C)

    partials = _build_sc_message_pass()(feature, src3, dst3)

    rows_blk = 1000
    grid = (N_NODES // rows_blk,)
    out = pl.pallas_call(
        _tc_linear_body,
        grid=grid,
        in_specs=[
            pl.BlockSpec((NC, rows_blk, D), lambda i: (0, i, 0)),
            pl.BlockSpec((D, D), lambda i: (0, 0)),
            pl.BlockSpec((1, D), lambda i: (0, 0)),
        ],
        out_specs=pl.BlockSpec((rows_blk, D), lambda i: (i, 0)),
        out_shape=jax.ShapeDtypeStruct((N_NODES, D), jnp.float32),
    )(partials, W, b.reshape(1, D))
    return out
